# Initial kernel scaffold; baseline (speedup 1.0000x reference)
import functools

import jax
import jax.numpy as jnp
from jax import lax
from jax.experimental import pallas as pl
from jax.experimental.pallas import tpu as pltpu
from jax.experimental.pallas import tpu_sc as plsc

N = 10000
E = 160000
F = 128
BE = 16
D = 64
H = 128
N2 = 20000
A = 40000
E2 = 160000
G = 256
NI2 = 64
M1_IN, M1_OUT = F, D // 2
M2_IN, M2_OUT = D // 2, D

NC = 2
NS = 16
NW = NC * NS
LCH = 128

EP = 163840
NCH_E = EP // (NW * LCH)
AP = 40960
NCH_A = AP // (NW * LCH)
NP1 = 10240
NP2 = 20480

EB = 1280
NBLK = EP // EB
NBLK_VALID = E // EB


def _sc_mesh():
    return plsc.VectorSubcoreMesh(core_axis_name="c", subcore_axis_name="s")


@functools.partial(jax.jit, static_argnames=("nch", "width"))
def _sc_gather(table, idx3, *, nch, width):
    nrows = NW * nch * LCH

    def body(table_ref, idx_ref, out_ref, idx_v, rows_v):
        c = lax.axis_index("c")
        s = lax.axis_index("s")
        wid = s * NC + c
        pltpu.sync_copy(idx_ref.at[wid], idx_v)

        def chunk(j, carry):
            pltpu.sync_copy(table_ref.at[idx_v.at[j]], rows_v)
            pltpu.sync_copy(rows_v, out_ref.at[pl.ds(wid * nch * LCH + j * LCH, LCH)])
            return carry

        lax.fori_loop(0, nch, chunk, 0)

    return pl.kernel(
        body,
        out_type=jax.ShapeDtypeStruct((nrows, width), jnp.float32),
        mesh=_sc_mesh(),
        scratch_types=[
            pltpu.VMEM((nch, LCH), jnp.int32),
            pltpu.VMEM((LCH, width), jnp.float32),
        ],
    )(table, idx3)


@functools.partial(jax.jit, static_argnames=("nch", "width", "npad"))
def _sc_scatter_add(data, idx3, zrow, *, nch, width, npad):
    nzc = npad // NS

    def body(data_ref, idx_ref, z_ref, out_ref, idx_v, rows_v, acc):
        c = lax.axis_index("c")
        s = lax.axis_index("s")
        wid = s * NC + c
        pltpu.sync_copy(idx_ref.at[wid], idx_v)
        pltpu.sync_copy(z_ref, acc.at[pl.ds(s * nzc, nzc)])
        plsc.subcore_barrier()

        def chunk(j, carry):
            pltpu.sync_copy(data_ref.at[pl.ds(wid * nch * LCH + j * LCH, LCH)], rows_v)
            pltpu.sync_copy(rows_v, acc.at[idx_v.at[j]], add=True)
            return carry

        lax.fori_loop(0, nch, chunk, 0)
        plsc.subcore_barrier()
        pltpu.sync_copy(acc.at[pl.ds(s * nzc, nzc)],
                        out_ref.at[pl.ds(c * npad + s * nzc, nzc)])

    return pl.kernel(
        body,
        out_type=jax.ShapeDtypeStruct((NC * npad, width), jnp.float32),
        mesh=_sc_mesh(),
        scratch_types=[
            pltpu.VMEM((nch, LCH), jnp.int32),
            pltpu.VMEM((LCH, width), jnp.float32),
            pltpu.VMEM_SHARED((npad, width), jnp.float32),
        ],
    )(data, idx3, zrow)


@functools.partial(jax.jit, static_argnames=("nch", "width", "npad"))
def _sc_gather_scatter(table, gidx3, sidx3, zrow, *, nch, width, npad):
    nzc = npad // NS

    def body(table_ref, gidx_ref, sidx_ref, z_ref, out_ref,
             gidx_v, sidx_v, rows_v, acc):
        c = lax.axis_index("c")
        s = lax.axis_index("s")
        wid = s * NC + c
        pltpu.sync_copy(gidx_ref.at[wid], gidx_v)
        pltpu.sync_copy(sidx_ref.at[wid], sidx_v)
        pltpu.sync_copy(z_ref, acc.at[pl.ds(s * nzc, nzc)])
        plsc.subcore_barrier()

        def chunk(j, carry):
            pltpu.sync_copy(table_ref.at[gidx_v.at[j]], rows_v)
            pltpu.sync_copy(rows_v, acc.at[sidx_v.at[j]], add=True)
            return carry

        lax.fori_loop(0, nch, chunk, 0)
        plsc.subcore_barrier()
        pltpu.sync_copy(acc.at[pl.ds(s * nzc, nzc)],
                        out_ref.at[pl.ds(c * npad + s * nzc, nzc)])

    return pl.kernel(
        body,
        out_type=jax.ShapeDtypeStruct((NC * npad, width), jnp.float32),
        mesh=_sc_mesh(),
        scratch_types=[
            pltpu.VMEM((nch, LCH), jnp.int32),
            pltpu.VMEM((nch, LCH), jnp.int32),
            pltpu.VMEM((LCH, width), jnp.float32),
            pltpu.VMEM_SHARED((npad, width), jnp.float32),
        ],
    )(table, gidx3, sidx3, zrow)


@jax.jit
def _sc_pool2(table, gidx3, sidx3, ones_in, z64, z16):
    nch = NCH_A
    npad = NP2
    nzc = npad // NS

    def body(table_ref, gidx_ref, sidx_ref, ones_ref, z64_ref, z16_ref,
             sum_ref, cnt_ref, gidx_v, sidx_v, rows_v, ones_v, acc, accc):
        c = lax.axis_index("c")
        s = lax.axis_index("s")
        wid = s * NC + c
        pltpu.sync_copy(gidx_ref.at[wid], gidx_v)
        pltpu.sync_copy(sidx_ref.at[wid], sidx_v)
        pltpu.sync_copy(ones_ref, ones_v)
        pltpu.sync_copy(z64_ref, acc.at[pl.ds(s * nzc, nzc)])
        pltpu.sync_copy(z16_ref, accc.at[pl.ds(s * nzc, nzc)])
        plsc.subcore_barrier()

        def chunk(j, carry):
            pltpu.sync_copy(table_ref.at[gidx_v.at[j]], rows_v)
            pltpu.sync_copy(rows_v, acc.at[sidx_v.at[j]], add=True)
            pltpu.sync_copy(ones_v, accc.at[sidx_v.at[j]], add=True)
            return carry

        lax.fori_loop(0, nch, chunk, 0)
        plsc.subcore_barrier()
        pltpu.sync_copy(acc.at[pl.ds(s * nzc, nzc)],
                        sum_ref.at[pl.ds(c * npad + s * nzc, nzc)])
        pltpu.sync_copy(accc.at[pl.ds(s * nzc, nzc)],
                        cnt_ref.at[pl.ds(c * npad + s * nzc, nzc)])

    return pl.kernel(
        body,
        out_type=(jax.ShapeDtypeStruct((NC * npad, D), jnp.float32),
                  jax.ShapeDtypeStruct((NC * npad, 16), jnp.float32)),
        mesh=_sc_mesh(),
        scratch_types=[
            pltpu.VMEM((nch, LCH), jnp.int32),
            pltpu.VMEM((nch, LCH), jnp.int32),
            pltpu.VMEM((LCH, D), jnp.float32),
            pltpu.VMEM((LCH, 16), jnp.float32),
            pltpu.VMEM_SHARED((npad, D), jnp.float32),
            pltpu.VMEM_SHARED((npad, 16), jnp.float32),
        ],
    )(table, gidx3, sidx3, ones_in, z64, z16)


def _dot(a, b):
    return jnp.dot(a, b, preferred_element_type=jnp.float32)


def _edge1_body(ea_ref, xs_ref, wa_ref, ba_ref, wr_ref, br_ref, out_ref):
    i = pl.program_id(0)

    @pl.when(i < NBLK_VALID)
    def _():
        h = jnp.maximum(_dot(ea_ref[...], wa_ref[...]) + ba_ref[...], 0.0)
        u = _dot(h, wr_ref[...])
        u3 = u.reshape(EB, M1_OUT, M1_IN) + br_ref[...][None]
        msg = jnp.sum(u3 * xs_ref[...][:, None, :], axis=2)
        out_ref[...] = msg

    @pl.when(i >= NBLK_VALID)
    def _():
        out_ref[...] = jnp.zeros_like(out_ref)


@jax.jit
def _edge1(ea, xs, wa, ba, wr, br):
    return pl.pallas_call(
        _edge1_body,
        grid=(NBLK,),
        in_specs=[
            pl.BlockSpec((EB, BE), lambda i: (i, 0)),
            pl.BlockSpec((EB, M1_IN), lambda i: (i, 0)),
            pl.BlockSpec((BE, H), lambda i: (0, 0)),
            pl.BlockSpec((1, H), lambda i: (0, 0)),
            pl.BlockSpec((H, M1_OUT * M1_IN), lambda i: (0, 0)),
            pl.BlockSpec((M1_OUT, M1_IN), lambda i: (0, 0)),
        ],
        out_specs=pl.BlockSpec((EB, M1_OUT), lambda i: (i, 0)),
        out_shape=jax.ShapeDtypeStruct((EP, M1_OUT), jnp.float32),
    )(ea, xs, wa, ba, wr, br)


def _edge2_body(ea_ref, xs_ref, wa_ref, ba_ref, wr_ref, br_ref, out_ref):
    i = pl.program_id(0)

    @pl.when(i < NBLK_VALID)
    def _():
        h = jnp.maximum(_dot(ea_ref[...], wa_ref[...]) + ba_ref[...], 0.0)
        u = _dot(h, wr_ref[...])
        u3 = u.reshape(EB, M2_IN, 128) + br_ref[...][None]
        msgp = jnp.sum(u3 * xs_ref[...][:, :, None], axis=1)
        out_ref[...] = msgp[:, :M2_OUT]

    @pl.when(i >= NBLK_VALID)
    def _():
        out_ref[...] = jnp.zeros_like(out_ref)


@jax.jit
def _edge2(ea, xs, wa, ba, wr, br):
    return pl.pallas_call(
        _edge2_body,
        grid=(NBLK,),
        in_specs=[
            pl.BlockSpec((EB, BE), lambda i: (i, 0)),
            pl.BlockSpec((EB, M2_IN), lambda i: (i, 0)),
            pl.BlockSpec((BE, H), lambda i: (0, 0)),
            pl.BlockSpec((1, H), lambda i: (0, 0)),
            pl.BlockSpec((H, M2_IN * 128), lambda i: (0, 0)),
            pl.BlockSpec((M2_IN, 128), lambda i: (0, 0)),
        ],
        out_specs=pl.BlockSpec((EB, M2_OUT), lambda i: (i, 0)),
        out_shape=jax.ShapeDtypeStruct((EP, M2_OUT), jnp.float32),
    )(ea, xs, wa, ba, wr, br)


def _node1_body(x_ref, wr_ref, agg_ref, b_ref, out_ref):
    agg = agg_ref[0:N, :] + agg_ref[NP1:NP1 + N, :]
    out_ref[...] = jnp.maximum(_dot(x_ref[...], wr_ref[...]) + agg + b_ref[...], 0.0)


@jax.jit
def _node1(x, wr, agg, b):
    return pl.pallas_call(
        _node1_body,
        out_shape=jax.ShapeDtypeStruct((N, M1_OUT), jnp.float32),
    )(x, wr, agg, b)


def _node2_body(x_ref, wr_ref, agg_ref, b_ref, batch_ref, h_ref, x1_ref):
    agg = agg_ref[0:N, :] + agg_ref[NP1:NP1 + N, :]
    h = jnp.maximum(_dot(x_ref[...], wr_ref[...]) + agg + b_ref[...], 0.0)
    h_ref[...] = h
    seg = lax.broadcasted_iota(jnp.int32, (G, N), 0)
    oh = (seg == batch_ref[...]).astype(jnp.float32)
    s = _dot(oh, h)
    cnt = jnp.sum(oh, axis=1, keepdims=True)
    x1_ref[...] = s / jnp.maximum(cnt, 1.0)


@jax.jit
def _node2(x, wr, agg, b, batch2d):
    return pl.pallas_call(
        _node2_body,
        out_shape=(jax.ShapeDtypeStruct((N, M2_OUT), jnp.float32),
                   jax.ShapeDtypeStruct((G, M2_OUT), jnp.float32)),
    )(x, wr, agg, b, batch2d)


def _convpre_body(hs_ref, hc_ref, iso_ref, wn_ref, ws_ref, b_ref, t_ref, s_ref):
    ssum = hs_ref[0:N2, :] + hs_ref[NP2:NP2 + N2, :]
    cnt = hc_ref[0:N2, 0:1] + hc_ref[NP2:NP2 + N2, 0:1]
    mean = ssum / jnp.maximum(cnt, 1.0)
    hc = jnp.concatenate([mean, iso_ref[...]], axis=1)
    t_ref[...] = _dot(hc, wn_ref[...])
    s_ref[...] = _dot(hc, ws_ref[...]) + b_ref[...]


@jax.jit
def _convpre(hsum, hcnt, iso, wn, ws, b):
    return pl.pallas_call(
        _convpre_body,
        out_shape=(jax.ShapeDtypeStruct((N2, D), jnp.float32),
                   jax.ShapeDtypeStruct((N2, D), jnp.float32)),
    )(hsum, hcnt, iso, wn, ws, b)


def _convmid_body(s_ref, agg_ref, wn_ref, ws_ref, b_ref, t_ref, so_ref):
    agg = agg_ref[0:N2, :] + agg_ref[NP2:NP2 + N2, :]
    h = jnp.maximum(s_ref[...] + agg, 0.0)
    t_ref[...] = _dot(h, wn_ref[...])
    so_ref[...] = _dot(h, ws_ref[...]) + b_ref[...]


@jax.jit
def _convmid(s4, agg, wn, ws, b):
    return pl.pallas_call(
        _convmid_body,
        out_shape=(jax.ShapeDtypeStruct((N2, D), jnp.float32),
                   jax.ShapeDtypeStruct((N2, D), jnp.float32)),
    )(s4, agg, wn, ws, b)


def _final_body(s_ref, agg_ref, batch_ref, x1_ref,
                wo1_ref, bo1_ref, wo2_ref, bo2_ref, wo3_ref, bo3_ref, o_ref):
    agg = agg_ref[0:N2, :] + agg_ref[NP2:NP2 + N2, :]
    h = jnp.maximum(s_ref[...] + agg, 0.0)
    seg = lax.broadcasted_iota(jnp.int32, (G, N2), 0)
    oh = (seg == batch_ref[...]).astype(jnp.float32)
    ssum = _dot(oh, h)
    cnt = jnp.sum(oh, axis=1, keepdims=True)
    x2 = ssum / jnp.maximum(cnt, 1.0)
    m = jnp.concatenate([x1_ref[...], x2], axis=1)
    o = jnp.maximum(_dot(m, wo1_ref[...]) + bo1_ref[...], 0.0)
    o = jnp.maximum(_dot(o, wo2_ref[...]) + bo2_ref[...], 0.0)
    o_ref[...] = _dot(o, wo3_ref[...]) + bo3_ref[...]


@jax.jit
def _final(s5, agg, batch2d, x1, wo1, bo1, wo2, bo2, wo3, bo3):
    return pl.pallas_call(
        _final_body,
        out_shape=jax.ShapeDtypeStruct((G, 1), jnp.float32),
    )(s5, agg, batch2d, x1, wo1, bo1, wo2, bo2, wo3, bo3)


def kernel(x, edge_index, edge_attr, batch, assignment_index_2, iso_type_2,
           edge_index_2, batch_2,
           We1a, be1a, We1b, be1b, Wr1, b1, We2a, be2a, We2b, be2b, Wr2, b2,
           W4s, W4n, b4, W5s, W5n, b5, Wo1, bo1, Wo2, bo2, Wo3, bo3):
    i32 = jnp.int32
    src = edge_index[0].astype(i32)
    dst = edge_index[1].astype(i32)
    src_p = jnp.concatenate([src, jnp.zeros((EP - E,), i32)]).reshape(NW, NCH_E, LCH)
    dst_p = jnp.concatenate([dst, jnp.full((EP - E,), N, i32)]).reshape(NW, NCH_E, LCH)
    ea_p = jnp.concatenate([edge_attr, jnp.zeros((EP - E, BE), jnp.float32)])

    src2 = edge_index_2[0].astype(i32)
    dst2 = edge_index_2[1].astype(i32)
    src2_p = jnp.concatenate([src2, jnp.zeros((EP - E2,), i32)]).reshape(NW, NCH_E, LCH)
    dst2_p = jnp.concatenate([dst2, jnp.full((EP - E2,), N2, i32)]).reshape(NW, NCH_E, LCH)

    a0 = assignment_index_2[0].astype(i32)
    a1 = assignment_index_2[1].astype(i32)
    a0_p = jnp.concatenate([a0, jnp.zeros((AP - A,), i32)]).reshape(NW, NCH_A, LCH)
    a1_p = jnp.concatenate([a1, jnp.full((AP - A,), N2, i32)]).reshape(NW, NCH_A, LCH)

    W1r = We1b.reshape(H, M1_IN, M1_OUT).transpose(0, 2, 1).reshape(H, M1_OUT * M1_IN)
    b1r = be1b.reshape(M1_IN, M1_OUT).T
    w23 = We2b.reshape(H, M2_IN, M2_OUT)
    W2r = jnp.pad(w23, ((0, 0), (0, 0), (0, 128 - M2_OUT))).reshape(H, M2_IN * 128)
    b2r = jnp.pad(be2b.reshape(M2_IN, M2_OUT), ((0, 0), (0, 128 - M2_OUT)))

    be1a2 = be1a.reshape(1, H)
    be2a2 = be2a.reshape(1, H)
    b1_2 = b1.reshape(1, M1_OUT)
    b2_2 = b2.reshape(1, M2_OUT)
    b4_2 = b4.reshape(1, D)
    b5_2 = b5.reshape(1, D)
    bo1_2 = bo1.reshape(1, D)
    bo2_2 = bo2.reshape(1, D // 2)
    bo3_2 = bo3.reshape(1, 1)
    batch2d = batch.astype(i32).reshape(1, N)
    batch2_2d = batch_2.astype(i32).reshape(1, N2)

    z1 = jnp.zeros((NP1 // NS, M1_OUT), jnp.float32)
    z2 = jnp.zeros((NP1 // NS, M2_OUT), jnp.float32)
    z64 = jnp.zeros((NP2 // NS, D), jnp.float32)
    z16 = jnp.zeros((NP2 // NS, 16), jnp.float32)
    ones16 = jnp.ones((LCH, 16), jnp.float32)

    x_src = _sc_gather(x, src_p, nch=NCH_E, width=F)
    msg1 = _edge1(ea_p, x_src, We1a, be1a2, W1r, b1r)
    agg1 = _sc_scatter_add(msg1, dst_p, z1, nch=NCH_E, width=M1_OUT, npad=NP1)
    out1 = _node1(x, Wr1, agg1, b1_2)

    x_src2 = _sc_gather(out1, src_p, nch=NCH_E, width=M1_OUT)
    msg2 = _edge2(ea_p, x_src2, We2a, be2a2, W2r, b2r)
    agg2 = _sc_scatter_add(msg2, dst_p, z2, nch=NCH_E, width=M2_OUT, npad=NP1)
    h, x_1 = _node2(out1, Wr2, agg2, b2_2, batch2d)

    hsum, hcnt = _sc_pool2(h, a0_p, a1_p, ones16, z64, z16)

    t4, s4 = _convpre(hsum, hcnt, iso_type_2, W4n, W4s, b4_2)
    agg4 = _sc_gather_scatter(t4, src2_p, dst2_p, z64, nch=NCH_E, width=D, npad=NP2)
    t5, s5 = _convmid(s4, agg4, W5n, W5s, b5_2)
    agg5 = _sc_gather_scatter(t5, src2_p, dst2_p, z64, nch=NCH_E, width=D, npad=NP2)

    o = _final(s5, agg5, batch2_2d, x_1, Wo1, bo1_2, Wo2, bo2_2, Wo3, bo3_2)
    return o.reshape(-1)


# speedup vs baseline: 2.4194x; 2.4194x over previous
import functools

import jax
import jax.numpy as jnp
from jax import lax
from jax.experimental import pallas as pl
from jax.experimental.pallas import tpu as pltpu
from jax.experimental.pallas import tpu_sc as plsc

N = 10000
E = 160000
F = 128
BE = 16
D = 64
H = 128
N2 = 20000
A = 40000
E2 = 160000
G = 256
NI2 = 64
M1_IN, M1_OUT = F, D // 2
M2_IN, M2_OUT = D // 2, D

NC = 2
NS = 16
NW = NC * NS
LCH = 128

EP = 163840
NCH_E = EP // (NW * LCH)
AP = 40960
NCH_A = AP // (NW * LCH)
NP1 = 10240
NP2 = 20480

EB = 640
NBLK = EP // EB
NBLK_VALID = E // EB
PB = 2000


def _sc_mesh():
    return plsc.VectorSubcoreMesh(core_axis_name="c", subcore_axis_name="s")


@functools.partial(jax.jit, static_argnames=("nch", "width"))
def _sc_gather(table, idx3, *, nch, width):
    nrows = NW * nch * LCH

    def body(table_ref, idx_ref, out_ref, idx_v, rows_v):
        c = lax.axis_index("c")
        s = lax.axis_index("s")
        wid = s * NC + c
        pltpu.sync_copy(idx_ref.at[wid], idx_v)

        def chunk(j, carry):
            pltpu.sync_copy(table_ref.at[idx_v.at[j]], rows_v)
            pltpu.sync_copy(rows_v, out_ref.at[pl.ds(wid * nch * LCH + j * LCH, LCH)])
            return carry

        lax.fori_loop(0, nch, chunk, 0)

    return pl.kernel(
        body,
        out_type=jax.ShapeDtypeStruct((nrows, width), jnp.float32),
        mesh=_sc_mesh(),
        compiler_params=pltpu.CompilerParams(use_tc_tiling_on_sc=False),
        scratch_types=[
            pltpu.VMEM((nch, LCH), jnp.int32),
            pltpu.VMEM((LCH, width), jnp.float32),
        ],
    )(table, idx3)


@functools.partial(jax.jit, static_argnames=("nch", "width", "npad"))
def _sc_scatter_add(data, idx3, zrow, *, nch, width, npad):
    nzc = npad // NS

    def body(data_ref, idx_ref, z_ref, out_ref, idx_v, rows_v, acc):
        c = lax.axis_index("c")
        s = lax.axis_index("s")
        wid = s * NC + c
        pltpu.sync_copy(idx_ref.at[wid], idx_v)
        pltpu.sync_copy(z_ref, acc.at[pl.ds(s * nzc, nzc)])
        plsc.subcore_barrier()

        def chunk(j, carry):
            pltpu.sync_copy(data_ref.at[pl.ds(wid * nch * LCH + j * LCH, LCH)], rows_v)
            pltpu.sync_copy(rows_v, acc.at[idx_v.at[j]], add=True)
            return carry

        lax.fori_loop(0, nch, chunk, 0)
        plsc.subcore_barrier()
        pltpu.sync_copy(acc.at[pl.ds(s * nzc, nzc)],
                        out_ref.at[pl.ds(c * npad + s * nzc, nzc)])

    return pl.kernel(
        body,
        out_type=jax.ShapeDtypeStruct((NC * npad, width), jnp.float32),
        mesh=_sc_mesh(),
        compiler_params=pltpu.CompilerParams(use_tc_tiling_on_sc=False),
        scratch_types=[
            pltpu.VMEM((nch, LCH), jnp.int32),
            pltpu.VMEM((LCH, width), jnp.float32),
            pltpu.VMEM_SHARED((npad, width), jnp.float32),
        ],
    )(data, idx3, zrow)


@functools.partial(jax.jit, static_argnames=("nch", "width", "npad"))
def _sc_gather_scatter(table, gidx3, sidx3, zrow, *, nch, width, npad):
    nzc = npad // NS

    def body(table_ref, gidx_ref, sidx_ref, z_ref, out_ref,
             gidx_v, sidx_v, rows_v, acc):
        c = lax.axis_index("c")
        s = lax.axis_index("s")
        wid = s * NC + c
        pltpu.sync_copy(gidx_ref.at[wid], gidx_v)
        pltpu.sync_copy(sidx_ref.at[wid], sidx_v)
        pltpu.sync_copy(z_ref, acc.at[pl.ds(s * nzc, nzc)])
        plsc.subcore_barrier()

        def chunk(j, carry):
            pltpu.sync_copy(table_ref.at[gidx_v.at[j]], rows_v)
            pltpu.sync_copy(rows_v, acc.at[sidx_v.at[j]], add=True)
            return carry

        lax.fori_loop(0, nch, chunk, 0)
        plsc.subcore_barrier()
        pltpu.sync_copy(acc.at[pl.ds(s * nzc, nzc)],
                        out_ref.at[pl.ds(c * npad + s * nzc, nzc)])

    return pl.kernel(
        body,
        out_type=jax.ShapeDtypeStruct((NC * npad, width), jnp.float32),
        mesh=_sc_mesh(),
        compiler_params=pltpu.CompilerParams(use_tc_tiling_on_sc=False),
        scratch_types=[
            pltpu.VMEM((nch, LCH), jnp.int32),
            pltpu.VMEM((nch, LCH), jnp.int32),
            pltpu.VMEM((LCH, width), jnp.float32),
            pltpu.VMEM_SHARED((npad, width), jnp.float32),
        ],
    )(table, gidx3, sidx3, zrow)


@jax.jit
def _sc_pool2(table, gidx3, sidx3, ones_in, z64, z16):
    nch = NCH_A
    npad = NP2
    nzc = npad // NS

    def body(table_ref, gidx_ref, sidx_ref, ones_ref, z64_ref, z16_ref,
             sum_ref, cnt_ref, gidx_v, sidx_v, rows_v, ones_v, acc, accc):
        c = lax.axis_index("c")
        s = lax.axis_index("s")
        wid = s * NC + c
        pltpu.sync_copy(gidx_ref.at[wid], gidx_v)
        pltpu.sync_copy(sidx_ref.at[wid], sidx_v)
        pltpu.sync_copy(ones_ref, ones_v)
        pltpu.sync_copy(z64_ref, acc.at[pl.ds(s * nzc, nzc)])
        pltpu.sync_copy(z16_ref, accc.at[pl.ds(s * nzc, nzc)])
        plsc.subcore_barrier()

        def chunk(j, carry):
            pltpu.sync_copy(table_ref.at[gidx_v.at[j]], rows_v)
            pltpu.sync_copy(rows_v, acc.at[sidx_v.at[j]], add=True)
            pltpu.sync_copy(ones_v, accc.at[sidx_v.at[j]], add=True)
            return carry

        lax.fori_loop(0, nch, chunk, 0)
        plsc.subcore_barrier()
        pltpu.sync_copy(acc.at[pl.ds(s * nzc, nzc)],
                        sum_ref.at[pl.ds(c * npad + s * nzc, nzc)])
        pltpu.sync_copy(accc.at[pl.ds(s * nzc, nzc)],
                        cnt_ref.at[pl.ds(c * npad + s * nzc, nzc)])

    return pl.kernel(
        body,
        out_type=(jax.ShapeDtypeStruct((NC * npad, D), jnp.float32),
                  jax.ShapeDtypeStruct((NC * npad, 16), jnp.float32)),
        mesh=_sc_mesh(),
        compiler_params=pltpu.CompilerParams(use_tc_tiling_on_sc=False),
        scratch_types=[
            pltpu.VMEM((nch, LCH), jnp.int32),
            pltpu.VMEM((nch, LCH), jnp.int32),
            pltpu.VMEM((LCH, D), jnp.float32),
            pltpu.VMEM((LCH, 16), jnp.float32),
            pltpu.VMEM_SHARED((npad, D), jnp.float32),
            pltpu.VMEM_SHARED((npad, 16), jnp.float32),
        ],
    )(table, gidx3, sidx3, ones_in, z64, z16)


def _dot(a, b):
    return jnp.dot(a, b, preferred_element_type=jnp.float32)


def _edge1_body(ea_ref, xs_ref, wa_ref, ba_ref, wr_ref, br_ref, out_ref):
    i = pl.program_id(0)

    @pl.when(i < NBLK_VALID)
    def _():
        h = jnp.maximum(_dot(ea_ref[...], wa_ref[...]) + ba_ref[...], 0.0)
        u = _dot(h, wr_ref[...])
        u3 = u.reshape(EB, M1_OUT, M1_IN) + br_ref[...][None]
        msg = jnp.sum(u3 * xs_ref[...][:, None, :], axis=2)
        out_ref[...] = msg

    @pl.when(i >= NBLK_VALID)
    def _():
        out_ref[...] = jnp.zeros_like(out_ref)


@jax.jit
def _edge1(ea, xs, wa, ba, wr, br):
    return pl.pallas_call(
        _edge1_body,
        grid=(NBLK,),
        in_specs=[
            pl.BlockSpec((EB, BE), lambda i: (i, 0)),
            pl.BlockSpec((EB, M1_IN), lambda i: (i, 0)),
            pl.BlockSpec((BE, H), lambda i: (0, 0)),
            pl.BlockSpec((1, H), lambda i: (0, 0)),
            pl.BlockSpec((H, M1_OUT * M1_IN), lambda i: (0, 0)),
            pl.BlockSpec((M1_OUT, M1_IN), lambda i: (0, 0)),
        ],
        out_specs=pl.BlockSpec((EB, M1_OUT), lambda i: (i, 0)),
        out_shape=jax.ShapeDtypeStruct((EP, M1_OUT), jnp.float32),
    )(ea, xs, wa, ba, wr, br)


def _edge2_body(ea_ref, xs_ref, wa_ref, ba_ref, wr_ref, br_ref, out_ref):
    i = pl.program_id(0)

    @pl.when(i < NBLK_VALID)
    def _():
        h = jnp.maximum(_dot(ea_ref[...], wa_ref[...]) + ba_ref[...], 0.0)
        u = _dot(h, wr_ref[...])
        u3 = u.reshape(EB, M2_IN, 128) + br_ref[...][None]
        msgp = jnp.sum(u3 * xs_ref[...][:, :, None], axis=1)
        out_ref[...] = msgp[:, :M2_OUT]

    @pl.when(i >= NBLK_VALID)
    def _():
        out_ref[...] = jnp.zeros_like(out_ref)


@jax.jit
def _edge2(ea, xs, wa, ba, wr, br):
    return pl.pallas_call(
        _edge2_body,
        grid=(NBLK,),
        in_specs=[
            pl.BlockSpec((EB, BE), lambda i: (i, 0)),
            pl.BlockSpec((EB, M2_IN), lambda i: (i, 0)),
            pl.BlockSpec((BE, H), lambda i: (0, 0)),
            pl.BlockSpec((1, H), lambda i: (0, 0)),
            pl.BlockSpec((H, M2_IN * 128), lambda i: (0, 0)),
            pl.BlockSpec((M2_IN, 128), lambda i: (0, 0)),
        ],
        out_specs=pl.BlockSpec((EB, M2_OUT), lambda i: (i, 0)),
        out_shape=jax.ShapeDtypeStruct((EP, M2_OUT), jnp.float32),
    )(ea, xs, wa, ba, wr, br)


def _node1_body(x_ref, wr_ref, agg_ref, b_ref, out_ref):
    agg = agg_ref[0:N, :] + agg_ref[NP1:NP1 + N, :]
    out_ref[...] = jnp.maximum(_dot(x_ref[...], wr_ref[...]) + agg + b_ref[...], 0.0)


@jax.jit
def _node1(x, wr, agg, b):
    return pl.pallas_call(
        _node1_body,
        out_shape=jax.ShapeDtypeStruct((N, M1_OUT), jnp.float32),
    )(x, wr, agg, b)


def _node2_body(x_ref, wr_ref, agg_ref, b_ref, batch_ref, h_ref, x1_ref):
    agg = agg_ref[0:N, :] + agg_ref[NP1:NP1 + N, :]
    h = jnp.maximum(_dot(x_ref[...], wr_ref[...]) + agg + b_ref[...], 0.0)
    h_ref[...] = h
    seg = lax.broadcasted_iota(jnp.int32, (G, N), 0)
    oh = (seg == batch_ref[...]).astype(jnp.float32)
    s = _dot(oh, h)
    cnt = jnp.sum(oh, axis=1, keepdims=True)
    x1_ref[...] = s / jnp.maximum(cnt, 1.0)


@jax.jit
def _node2(x, wr, agg, b, batch2d):
    return pl.pallas_call(
        _node2_body,
        out_shape=(jax.ShapeDtypeStruct((N, M2_OUT), jnp.float32),
                   jax.ShapeDtypeStruct((G, M2_OUT), jnp.float32)),
    )(x, wr, agg, b, batch2d)


def _convpre_body(hs0_ref, hs1_ref, hc0_ref, hc1_ref, iso_ref, wn_ref, ws_ref,
                  b_ref, t_ref, s_ref):
    ssum = hs0_ref[...] + hs1_ref[...]
    cnt = hc0_ref[...][:, 0:1] + hc1_ref[...][:, 0:1]
    mean = ssum / jnp.maximum(cnt, 1.0)
    hc = jnp.concatenate([mean, iso_ref[...]], axis=1)
    t_ref[...] = _dot(hc, wn_ref[...])
    s_ref[...] = _dot(hc, ws_ref[...]) + b_ref[...]


@jax.jit
def _convpre(hs0, hs1, hc0, hc1, iso, wn, ws, b):
    nblk = N2 // PB
    row = lambda j: (j, 0)
    const = lambda j: (0, 0)
    return pl.pallas_call(
        _convpre_body,
        grid=(nblk,),
        in_specs=[
            pl.BlockSpec((PB, D), row),
            pl.BlockSpec((PB, D), row),
            pl.BlockSpec((PB, 16), row),
            pl.BlockSpec((PB, 16), row),
            pl.BlockSpec((PB, NI2), row),
            pl.BlockSpec((D + NI2, D), const),
            pl.BlockSpec((D + NI2, D), const),
            pl.BlockSpec((1, D), const),
        ],
        out_specs=(pl.BlockSpec((PB, D), row), pl.BlockSpec((PB, D), row)),
        out_shape=(jax.ShapeDtypeStruct((N2, D), jnp.float32),
                   jax.ShapeDtypeStruct((N2, D), jnp.float32)),
    )(hs0, hs1, hc0, hc1, iso, wn, ws, b)


def _convmid_body(s_ref, agg_ref, wn_ref, ws_ref, b_ref, t_ref, so_ref):
    agg = agg_ref[0:N2, :] + agg_ref[NP2:NP2 + N2, :]
    h = jnp.maximum(s_ref[...] + agg, 0.0)
    t_ref[...] = _dot(h, wn_ref[...])
    so_ref[...] = _dot(h, ws_ref[...]) + b_ref[...]


@jax.jit
def _convmid(s4, agg, wn, ws, b):
    return pl.pallas_call(
        _convmid_body,
        out_shape=(jax.ShapeDtypeStruct((N2, D), jnp.float32),
                   jax.ShapeDtypeStruct((N2, D), jnp.float32)),
    )(s4, agg, wn, ws, b)


def _pool2b_body(s_ref, a0_ref, a1_ref, batch_ref, sum_ref, cnt_ref):
    j = pl.program_id(0)
    h = jnp.maximum(s_ref[...] + a0_ref[...] + a1_ref[...], 0.0)
    seg = lax.broadcasted_iota(jnp.int32, (G, PB), 0)
    oh = (seg == batch_ref[0]).astype(jnp.float32)
    ps = _dot(oh, h)
    pc = jnp.sum(oh, axis=1, keepdims=True)

    @pl.when(j == 0)
    def _():
        sum_ref[...] = ps
        cnt_ref[...] = pc

    @pl.when(j > 0)
    def _():
        sum_ref[...] += ps
        cnt_ref[...] += pc


@jax.jit
def _pool2b(s5, agg0, agg1, batch2d):
    nblk = N2 // PB
    return pl.pallas_call(
        _pool2b_body,
        grid=(nblk,),
        in_specs=[
            pl.BlockSpec((PB, D), lambda j: (j, 0)),
            pl.BlockSpec((PB, D), lambda j: (j, 0)),
            pl.BlockSpec((PB, D), lambda j: (j, 0)),
            pl.BlockSpec((1, 1, PB), lambda j: (j, 0, 0)),
        ],
        out_specs=(pl.BlockSpec((G, D), lambda j: (0, 0)),
                   pl.BlockSpec((G, 1), lambda j: (0, 0))),
        out_shape=(jax.ShapeDtypeStruct((G, D), jnp.float32),
                   jax.ShapeDtypeStruct((G, 1), jnp.float32)),
    )(s5, agg0, agg1, batch2d)


def _readout_body(x1_ref, xs_ref, xc_ref,
                  wo1_ref, bo1_ref, wo2_ref, bo2_ref, wo3_ref, bo3_ref, o_ref):
    x2 = xs_ref[...] / jnp.maximum(xc_ref[...], 1.0)
    m = jnp.concatenate([x1_ref[...], x2], axis=1)
    o = jnp.maximum(_dot(m, wo1_ref[...]) + bo1_ref[...], 0.0)
    o = jnp.maximum(_dot(o, wo2_ref[...]) + bo2_ref[...], 0.0)
    o_ref[...] = _dot(o, wo3_ref[...]) + bo3_ref[...]


@jax.jit
def _readout(x1, x2s, x2c, wo1, bo1, wo2, bo2, wo3, bo3):
    return pl.pallas_call(
        _readout_body,
        out_shape=jax.ShapeDtypeStruct((G, 1), jnp.float32),
    )(x1, x2s, x2c, wo1, bo1, wo2, bo2, wo3, bo3)


def kernel(x, edge_index, edge_attr, batch, assignment_index_2, iso_type_2,
           edge_index_2, batch_2,
           We1a, be1a, We1b, be1b, Wr1, b1, We2a, be2a, We2b, be2b, Wr2, b2,
           W4s, W4n, b4, W5s, W5n, b5, Wo1, bo1, Wo2, bo2, Wo3, bo3):
    i32 = jnp.int32
    src = edge_index[0].astype(i32)
    dst = edge_index[1].astype(i32)
    src_p = jnp.concatenate([src, jnp.zeros((EP - E,), i32)]).reshape(NW, NCH_E, LCH)
    dst_p = jnp.concatenate([dst, jnp.full((EP - E,), N, i32)]).reshape(NW, NCH_E, LCH)
    ea_p = jnp.concatenate([edge_attr, jnp.zeros((EP - E, BE), jnp.float32)])

    src2 = edge_index_2[0].astype(i32)
    dst2 = edge_index_2[1].astype(i32)
    src2_p = jnp.concatenate([src2, jnp.zeros((EP - E2,), i32)]).reshape(NW, NCH_E, LCH)
    dst2_p = jnp.concatenate([dst2, jnp.full((EP - E2,), N2, i32)]).reshape(NW, NCH_E, LCH)

    a0 = assignment_index_2[0].astype(i32)
    a1 = assignment_index_2[1].astype(i32)
    a0_p = jnp.concatenate([a0, jnp.zeros((AP - A,), i32)]).reshape(NW, NCH_A, LCH)
    a1_p = jnp.concatenate([a1, jnp.full((AP - A,), N2, i32)]).reshape(NW, NCH_A, LCH)

    W1r = We1b.reshape(H, M1_IN, M1_OUT).transpose(0, 2, 1).reshape(H, M1_OUT * M1_IN)
    b1r = be1b.reshape(M1_IN, M1_OUT).T
    w23 = We2b.reshape(H, M2_IN, M2_OUT)
    W2r = jnp.pad(w23, ((0, 0), (0, 0), (0, 128 - M2_OUT))).reshape(H, M2_IN * 128)
    b2r = jnp.pad(be2b.reshape(M2_IN, M2_OUT), ((0, 0), (0, 128 - M2_OUT)))

    be1a2 = be1a.reshape(1, H)
    be2a2 = be2a.reshape(1, H)
    b1_2 = b1.reshape(1, M1_OUT)
    b2_2 = b2.reshape(1, M2_OUT)
    b4_2 = b4.reshape(1, D)
    b5_2 = b5.reshape(1, D)
    bo1_2 = bo1.reshape(1, D)
    bo2_2 = bo2.reshape(1, D // 2)
    bo3_2 = bo3.reshape(1, 1)
    batch2d = batch.astype(i32).reshape(1, N)
    batch2_2d = batch_2.astype(i32).reshape(1, N2)

    z1 = jnp.zeros((NP1 // NS, M1_OUT), jnp.float32)
    z2 = jnp.zeros((NP1 // NS, M2_OUT), jnp.float32)
    z64 = jnp.zeros((NP2 // NS, D), jnp.float32)
    z16 = jnp.zeros((NP2 // NS, 16), jnp.float32)
    ones16 = jnp.ones((LCH, 16), jnp.float32)

    x_src = _sc_gather(x, src_p, nch=NCH_E, width=F)
    msg1 = _edge1(ea_p, x_src, We1a, be1a2, W1r, b1r)
    agg1 = _sc_scatter_add(msg1, dst_p, z1, nch=NCH_E, width=M1_OUT, npad=NP1)
    out1 = _node1(x, Wr1, agg1, b1_2)

    x_src2 = _sc_gather(out1, src_p, nch=NCH_E, width=M1_OUT)
    msg2 = _edge2(ea_p, x_src2, We2a, be2a2, W2r, b2r)
    agg2 = _sc_scatter_add(msg2, dst_p, z2, nch=NCH_E, width=M2_OUT, npad=NP1)
    h, x_1 = _node2(out1, Wr2, agg2, b2_2, batch2d)

    hsum, hcnt = _sc_pool2(h, a0_p, a1_p, ones16, z64, z16)

    t4, s4 = _convpre(hsum[0:N2], hsum[NP2:NP2 + N2], hcnt[0:N2],
                      hcnt[NP2:NP2 + N2], iso_type_2, W4n, W4s, b4_2)
    agg4 = _sc_gather_scatter(t4, src2_p, dst2_p, z64, nch=NCH_E, width=D, npad=NP2)
    t5, s5 = _convmid(s4, agg4, W5n, W5s, b5_2)
    agg5 = _sc_gather_scatter(t5, src2_p, dst2_p, z64, nch=NCH_E, width=D, npad=NP2)

    batch2_3d = batch_2.astype(i32).reshape(N2 // PB, 1, PB)
    x2s, x2c = _pool2b(s5, agg5[0:N2], agg5[NP2:NP2 + N2], batch2_3d)
    o = _readout(x_1, x2s, x2c, Wo1, bo1_2, Wo2, bo2_2, Wo3, bo3_2)
    return o.reshape(-1)


# speedup vs baseline: 3.4249x; 1.4156x over previous
import functools

import jax
import jax.numpy as jnp
from jax import lax
from jax.experimental import pallas as pl
from jax.experimental.pallas import tpu as pltpu
from jax.experimental.pallas import tpu_sc as plsc

N = 10000
E = 160000
F = 128
BE = 16
D = 64
H = 128
N2 = 20000
A = 40000
E2 = 160000
G = 256
NI2 = 64
M1_IN, M1_OUT = F, D // 2
M2_IN, M2_OUT = D // 2, D

NC = 2
NS = 16
NW = NC * NS
LCH = 128

EP = 163840
NCH_E = EP // (NW * LCH)
AP = 40960
NCH_A = AP // (NW * LCH)
NP1 = 10240
NP2 = 20480

EB = 640
NBLK = EP // EB
NBLK_VALID = E // EB
PB = 2000


def _sc_mesh():
    return plsc.VectorSubcoreMesh(core_axis_name="c", subcore_axis_name="s")


@functools.partial(jax.jit, static_argnames=("nch", "width"))
def _sc_gather(table, idx3, *, nch, width):
    nrows = NW * nch * LCH

    def body(table_ref, idx_ref, out_ref, idx_v, rows_v):
        c = lax.axis_index("c")
        s = lax.axis_index("s")
        wid = s * NC + c
        pltpu.sync_copy(idx_ref.at[wid], idx_v)

        def chunk(j, carry):
            pltpu.sync_copy(table_ref.at[idx_v.at[j]], rows_v)
            pltpu.sync_copy(rows_v, out_ref.at[pl.ds(wid * nch * LCH + j * LCH, LCH)])
            return carry

        lax.fori_loop(0, nch, chunk, 0)

    return pl.kernel(
        body,
        out_type=jax.ShapeDtypeStruct((nrows, width), jnp.float32),
        mesh=_sc_mesh(),
        compiler_params=pltpu.CompilerParams(use_tc_tiling_on_sc=False),
        scratch_types=[
            pltpu.VMEM((nch, LCH), jnp.int32),
            pltpu.VMEM((LCH, width), jnp.float32),
        ],
    )(table, idx3)


@functools.partial(jax.jit, static_argnames=("nch", "width", "npad"))
def _sc_scatter_add(data, idx3, zrow, *, nch, width, npad):
    nzc = npad // NS

    def body(data_ref, idx_ref, z_ref, out_ref, idx_v, rows_v, acc):
        c = lax.axis_index("c")
        s = lax.axis_index("s")
        wid = s * NC + c
        pltpu.sync_copy(idx_ref.at[wid], idx_v)
        pltpu.sync_copy(z_ref, acc.at[pl.ds(s * nzc, nzc)])
        plsc.subcore_barrier()

        def chunk(j, carry):
            pltpu.sync_copy(data_ref.at[pl.ds(wid * nch * LCH + j * LCH, LCH)], rows_v)
            pltpu.sync_copy(rows_v, acc.at[idx_v.at[j]], add=True)
            return carry

        lax.fori_loop(0, nch, chunk, 0)
        plsc.subcore_barrier()
        pltpu.sync_copy(acc.at[pl.ds(s * nzc, nzc)],
                        out_ref.at[pl.ds(c * npad + s * nzc, nzc)])

    return pl.kernel(
        body,
        out_type=jax.ShapeDtypeStruct((NC * npad, width), jnp.float32),
        mesh=_sc_mesh(),
        compiler_params=pltpu.CompilerParams(use_tc_tiling_on_sc=False),
        scratch_types=[
            pltpu.VMEM((nch, LCH), jnp.int32),
            pltpu.VMEM((LCH, width), jnp.float32),
            pltpu.VMEM_SHARED((npad, width), jnp.float32),
        ],
    )(data, idx3, zrow)


@functools.partial(jax.jit, static_argnames=("nch", "width", "npad"))
def _sc_gather_scatter(table, gidx3, sidx3, zrow, *, nch, width, npad):
    nzc = npad // NS

    def body(table_ref, gidx_ref, sidx_ref, z_ref, out_ref,
             gidx_v, sidx_v, rows_v, acc):
        c = lax.axis_index("c")
        s = lax.axis_index("s")
        wid = s * NC + c
        pltpu.sync_copy(gidx_ref.at[wid], gidx_v)
        pltpu.sync_copy(sidx_ref.at[wid], sidx_v)
        pltpu.sync_copy(z_ref, acc.at[pl.ds(s * nzc, nzc)])
        plsc.subcore_barrier()

        def chunk(j, carry):
            pltpu.sync_copy(table_ref.at[gidx_v.at[j]], rows_v)
            pltpu.sync_copy(rows_v, acc.at[sidx_v.at[j]], add=True)
            return carry

        lax.fori_loop(0, nch, chunk, 0)
        plsc.subcore_barrier()
        pltpu.sync_copy(acc.at[pl.ds(s * nzc, nzc)],
                        out_ref.at[pl.ds(c * npad + s * nzc, nzc)])

    return pl.kernel(
        body,
        out_type=jax.ShapeDtypeStruct((NC * npad, width), jnp.float32),
        mesh=_sc_mesh(),
        compiler_params=pltpu.CompilerParams(use_tc_tiling_on_sc=False),
        scratch_types=[
            pltpu.VMEM((nch, LCH), jnp.int32),
            pltpu.VMEM((nch, LCH), jnp.int32),
            pltpu.VMEM((LCH, width), jnp.float32),
            pltpu.VMEM_SHARED((npad, width), jnp.float32),
        ],
    )(table, gidx3, sidx3, zrow)


@jax.jit
def _sc_pool2(table, gidx3, sidx3, ones_in, z64, z16):
    nch = NCH_A
    npad = NP2
    nzc = npad // NS

    def body(table_ref, gidx_ref, sidx_ref, ones_ref, z64_ref, z16_ref,
             sum_ref, cnt_ref, gidx_v, sidx_v, rows_v, ones_v, acc, accc):
        c = lax.axis_index("c")
        s = lax.axis_index("s")
        wid = s * NC + c
        pltpu.sync_copy(gidx_ref.at[wid], gidx_v)
        pltpu.sync_copy(sidx_ref.at[wid], sidx_v)
        pltpu.sync_copy(ones_ref, ones_v)
        pltpu.sync_copy(z64_ref, acc.at[pl.ds(s * nzc, nzc)])
        pltpu.sync_copy(z16_ref, accc.at[pl.ds(s * nzc, nzc)])
        plsc.subcore_barrier()

        def chunk(j, carry):
            pltpu.sync_copy(table_ref.at[gidx_v.at[j]], rows_v)
            pltpu.sync_copy(rows_v, acc.at[sidx_v.at[j]], add=True)
            pltpu.sync_copy(ones_v, accc.at[sidx_v.at[j]], add=True)
            return carry

        lax.fori_loop(0, nch, chunk, 0)
        plsc.subcore_barrier()
        pltpu.sync_copy(acc.at[pl.ds(s * nzc, nzc)],
                        sum_ref.at[pl.ds(c * npad + s * nzc, nzc)])
        pltpu.sync_copy(accc.at[pl.ds(s * nzc, nzc)],
                        cnt_ref.at[pl.ds(c * npad + s * nzc, nzc)])

    return pl.kernel(
        body,
        out_type=(jax.ShapeDtypeStruct((NC * npad, D), jnp.float32),
                  jax.ShapeDtypeStruct((NC * npad, 16), jnp.float32)),
        mesh=_sc_mesh(),
        compiler_params=pltpu.CompilerParams(use_tc_tiling_on_sc=False),
        scratch_types=[
            pltpu.VMEM((nch, LCH), jnp.int32),
            pltpu.VMEM((nch, LCH), jnp.int32),
            pltpu.VMEM((LCH, D), jnp.float32),
            pltpu.VMEM((LCH, 16), jnp.float32),
            pltpu.VMEM_SHARED((npad, D), jnp.float32),
            pltpu.VMEM_SHARED((npad, 16), jnp.float32),
        ],
    )(table, gidx3, sidx3, ones_in, z64, z16)


def _dot(a, b):
    return jnp.dot(a, b, preferred_element_type=jnp.float32)


def _edge1_body(ea_ref, xs_ref, wa_ref, ba_ref, wr_ref, br_ref, s_ref, out_ref):
    i = pl.program_id(0)

    @pl.when(i < NBLK_VALID)
    def _():
        h = jnp.maximum(_dot(ea_ref[...], wa_ref[...]) + ba_ref[...], 0.0)
        u = _dot(h, wr_ref[...]) + br_ref[...]
        v = u * jnp.tile(xs_ref[...], (1, M1_OUT))
        out_ref[...] = _dot(v, s_ref[...])

    @pl.when(i >= NBLK_VALID)
    def _():
        out_ref[...] = jnp.zeros_like(out_ref)


@jax.jit
def _edge1(ea, xs, wa, ba, wr, br, s):
    return pl.pallas_call(
        _edge1_body,
        grid=(NBLK,),
        in_specs=[
            pl.BlockSpec((EB, BE), lambda i: (i, 0)),
            pl.BlockSpec((EB, M1_IN), lambda i: (i, 0)),
            pl.BlockSpec((BE, H), lambda i: (0, 0)),
            pl.BlockSpec((1, H), lambda i: (0, 0)),
            pl.BlockSpec((H, M1_OUT * M1_IN), lambda i: (0, 0)),
            pl.BlockSpec((1, M1_OUT * M1_IN), lambda i: (0, 0)),
            pl.BlockSpec((M1_OUT * M1_IN, M1_OUT), lambda i: (0, 0)),
        ],
        out_specs=pl.BlockSpec((EB, M1_OUT), lambda i: (i, 0)),
        out_shape=jax.ShapeDtypeStruct((EP, M1_OUT), jnp.float32),
    )(ea, xs, wa, ba, wr, br, s)


def _edge2_body(ea_ref, xs_ref, wa_ref, ba_ref, wr_ref, br_ref, s_ref, out_ref):
    i = pl.program_id(0)

    @pl.when(i < NBLK_VALID)
    def _():
        h = jnp.maximum(_dot(ea_ref[...], wa_ref[...]) + ba_ref[...], 0.0)
        u = _dot(h, wr_ref[...]) + br_ref[...]
        v = u * jnp.tile(xs_ref[...], (1, M2_OUT))
        out_ref[...] = _dot(v, s_ref[...])

    @pl.when(i >= NBLK_VALID)
    def _():
        out_ref[...] = jnp.zeros_like(out_ref)


@jax.jit
def _edge2(ea, xs, wa, ba, wr, br, s):
    return pl.pallas_call(
        _edge2_body,
        grid=(NBLK,),
        in_specs=[
            pl.BlockSpec((EB, BE), lambda i: (i, 0)),
            pl.BlockSpec((EB, M2_IN), lambda i: (i, 0)),
            pl.BlockSpec((BE, H), lambda i: (0, 0)),
            pl.BlockSpec((1, H), lambda i: (0, 0)),
            pl.BlockSpec((H, M2_OUT * M2_IN), lambda i: (0, 0)),
            pl.BlockSpec((1, M2_OUT * M2_IN), lambda i: (0, 0)),
            pl.BlockSpec((M2_OUT * M2_IN, M2_OUT), lambda i: (0, 0)),
        ],
        out_specs=pl.BlockSpec((EB, M2_OUT), lambda i: (i, 0)),
        out_shape=jax.ShapeDtypeStruct((EP, M2_OUT), jnp.float32),
    )(ea, xs, wa, ba, wr, br, s)


def _node1_body(x_ref, wr_ref, agg_ref, b_ref, out_ref):
    agg = agg_ref[0:N, :] + agg_ref[NP1:NP1 + N, :]
    out_ref[...] = jnp.maximum(_dot(x_ref[...], wr_ref[...]) + agg + b_ref[...], 0.0)


@jax.jit
def _node1(x, wr, agg, b):
    return pl.pallas_call(
        _node1_body,
        out_shape=jax.ShapeDtypeStruct((N, M1_OUT), jnp.float32),
    )(x, wr, agg, b)


def _node2_body(x_ref, wr_ref, agg_ref, b_ref, batch_ref, h_ref, x1_ref):
    agg = agg_ref[0:N, :] + agg_ref[NP1:NP1 + N, :]
    h = jnp.maximum(_dot(x_ref[...], wr_ref[...]) + agg + b_ref[...], 0.0)
    h_ref[...] = h
    seg = lax.broadcasted_iota(jnp.int32, (G, N), 0)
    oh = (seg == batch_ref[...]).astype(jnp.float32)
    s = _dot(oh, h)
    cnt = jnp.sum(oh, axis=1, keepdims=True)
    x1_ref[...] = s / jnp.maximum(cnt, 1.0)


@jax.jit
def _node2(x, wr, agg, b, batch2d):
    return pl.pallas_call(
        _node2_body,
        out_shape=(jax.ShapeDtypeStruct((N, M2_OUT), jnp.float32),
                   jax.ShapeDtypeStruct((G, M2_OUT), jnp.float32)),
    )(x, wr, agg, b, batch2d)


def _convpre_body(hs0_ref, hs1_ref, hc0_ref, hc1_ref, iso_ref, wn_ref, ws_ref,
                  b_ref, t_ref, s_ref):
    ssum = hs0_ref[...] + hs1_ref[...]
    cnt = hc0_ref[...][:, 0:1] + hc1_ref[...][:, 0:1]
    mean = ssum / jnp.maximum(cnt, 1.0)
    hc = jnp.concatenate([mean, iso_ref[...]], axis=1)
    t_ref[...] = _dot(hc, wn_ref[...])
    s_ref[...] = _dot(hc, ws_ref[...]) + b_ref[...]


@jax.jit
def _convpre(hs0, hs1, hc0, hc1, iso, wn, ws, b):
    nblk = N2 // PB
    row = lambda j: (j, 0)
    const = lambda j: (0, 0)
    return pl.pallas_call(
        _convpre_body,
        grid=(nblk,),
        in_specs=[
            pl.BlockSpec((PB, D), row),
            pl.BlockSpec((PB, D), row),
            pl.BlockSpec((PB, 16), row),
            pl.BlockSpec((PB, 16), row),
            pl.BlockSpec((PB, NI2), row),
            pl.BlockSpec((D + NI2, D), const),
            pl.BlockSpec((D + NI2, D), const),
            pl.BlockSpec((1, D), const),
        ],
        out_specs=(pl.BlockSpec((PB, D), row), pl.BlockSpec((PB, D), row)),
        out_shape=(jax.ShapeDtypeStruct((N2, D), jnp.float32),
                   jax.ShapeDtypeStruct((N2, D), jnp.float32)),
    )(hs0, hs1, hc0, hc1, iso, wn, ws, b)


def _convmid_body(s_ref, agg_ref, wn_ref, ws_ref, b_ref, t_ref, so_ref):
    agg = agg_ref[0:N2, :] + agg_ref[NP2:NP2 + N2, :]
    h = jnp.maximum(s_ref[...] + agg, 0.0)
    t_ref[...] = _dot(h, wn_ref[...])
    so_ref[...] = _dot(h, ws_ref[...]) + b_ref[...]


@jax.jit
def _convmid(s4, agg, wn, ws, b):
    return pl.pallas_call(
        _convmid_body,
        out_shape=(jax.ShapeDtypeStruct((N2, D), jnp.float32),
                   jax.ShapeDtypeStruct((N2, D), jnp.float32)),
    )(s4, agg, wn, ws, b)


def _pool2b_body(s_ref, a0_ref, a1_ref, batch_ref, sum_ref, cnt_ref):
    j = pl.program_id(0)
    h = jnp.maximum(s_ref[...] + a0_ref[...] + a1_ref[...], 0.0)
    seg = lax.broadcasted_iota(jnp.int32, (G, PB), 0)
    oh = (seg == batch_ref[0]).astype(jnp.float32)
    ps = _dot(oh, h)
    pc = jnp.sum(oh, axis=1, keepdims=True)

    @pl.when(j == 0)
    def _():
        sum_ref[...] = ps
        cnt_ref[...] = pc

    @pl.when(j > 0)
    def _():
        sum_ref[...] += ps
        cnt_ref[...] += pc


@jax.jit
def _pool2b(s5, agg0, agg1, batch2d):
    nblk = N2 // PB
    return pl.pallas_call(
        _pool2b_body,
        grid=(nblk,),
        in_specs=[
            pl.BlockSpec((PB, D), lambda j: (j, 0)),
            pl.BlockSpec((PB, D), lambda j: (j, 0)),
            pl.BlockSpec((PB, D), lambda j: (j, 0)),
            pl.BlockSpec((1, 1, PB), lambda j: (j, 0, 0)),
        ],
        out_specs=(pl.BlockSpec((G, D), lambda j: (0, 0)),
                   pl.BlockSpec((G, 1), lambda j: (0, 0))),
        out_shape=(jax.ShapeDtypeStruct((G, D), jnp.float32),
                   jax.ShapeDtypeStruct((G, 1), jnp.float32)),
    )(s5, agg0, agg1, batch2d)


def _readout_body(x1_ref, xs_ref, xc_ref,
                  wo1_ref, bo1_ref, wo2_ref, bo2_ref, wo3_ref, bo3_ref, o_ref):
    x2 = xs_ref[...] / jnp.maximum(xc_ref[...], 1.0)
    m = jnp.concatenate([x1_ref[...], x2], axis=1)
    o = jnp.maximum(_dot(m, wo1_ref[...]) + bo1_ref[...], 0.0)
    o = jnp.maximum(_dot(o, wo2_ref[...]) + bo2_ref[...], 0.0)
    o_ref[...] = _dot(o, wo3_ref[...]) + bo3_ref[...]


@jax.jit
def _readout(x1, x2s, x2c, wo1, bo1, wo2, bo2, wo3, bo3):
    return pl.pallas_call(
        _readout_body,
        out_shape=jax.ShapeDtypeStruct((G, 1), jnp.float32),
    )(x1, x2s, x2c, wo1, bo1, wo2, bo2, wo3, bo3)


def kernel(x, edge_index, edge_attr, batch, assignment_index_2, iso_type_2,
           edge_index_2, batch_2,
           We1a, be1a, We1b, be1b, Wr1, b1, We2a, be2a, We2b, be2b, Wr2, b2,
           W4s, W4n, b4, W5s, W5n, b5, Wo1, bo1, Wo2, bo2, Wo3, bo3):
    i32 = jnp.int32
    src = edge_index[0].astype(i32)
    dst = edge_index[1].astype(i32)
    src_p = jnp.concatenate([src, jnp.zeros((EP - E,), i32)]).reshape(NW, NCH_E, LCH)
    dst_p = jnp.concatenate([dst, jnp.full((EP - E,), N, i32)]).reshape(NW, NCH_E, LCH)
    ea_p = jnp.concatenate([edge_attr, jnp.zeros((EP - E, BE), jnp.float32)])

    src2 = edge_index_2[0].astype(i32)
    dst2 = edge_index_2[1].astype(i32)
    src2_p = jnp.concatenate([src2, jnp.zeros((EP - E2,), i32)]).reshape(NW, NCH_E, LCH)
    dst2_p = jnp.concatenate([dst2, jnp.full((EP - E2,), N2, i32)]).reshape(NW, NCH_E, LCH)

    a0 = assignment_index_2[0].astype(i32)
    a1 = assignment_index_2[1].astype(i32)
    a0_p = jnp.concatenate([a0, jnp.zeros((AP - A,), i32)]).reshape(NW, NCH_A, LCH)
    a1_p = jnp.concatenate([a1, jnp.full((AP - A,), N2, i32)]).reshape(NW, NCH_A, LCH)

    W1r = We1b.reshape(H, M1_IN, M1_OUT).transpose(0, 2, 1).reshape(H, M1_OUT * M1_IN)
    b1r = be1b.reshape(M1_IN, M1_OUT).T.reshape(1, M1_OUT * M1_IN)
    W2r = We2b.reshape(H, M2_IN, M2_OUT).transpose(0, 2, 1).reshape(H, M2_OUT * M2_IN)
    b2r = be2b.reshape(M2_IN, M2_OUT).T.reshape(1, M2_OUT * M2_IN)
    S1 = jnp.repeat(jnp.eye(M1_OUT, dtype=jnp.float32), M1_IN, axis=0)
    S2 = jnp.repeat(jnp.eye(M2_OUT, dtype=jnp.float32), M2_IN, axis=0)

    be1a2 = be1a.reshape(1, H)
    be2a2 = be2a.reshape(1, H)
    b1_2 = b1.reshape(1, M1_OUT)
    b2_2 = b2.reshape(1, M2_OUT)
    b4_2 = b4.reshape(1, D)
    b5_2 = b5.reshape(1, D)
    bo1_2 = bo1.reshape(1, D)
    bo2_2 = bo2.reshape(1, D // 2)
    bo3_2 = bo3.reshape(1, 1)
    batch2d = batch.astype(i32).reshape(1, N)
    batch2_2d = batch_2.astype(i32).reshape(1, N2)

    z1 = jnp.zeros((NP1 // NS, M1_OUT), jnp.float32)
    z2 = jnp.zeros((NP1 // NS, M2_OUT), jnp.float32)
    z64 = jnp.zeros((NP2 // NS, D), jnp.float32)
    z16 = jnp.zeros((NP2 // NS, 16), jnp.float32)
    ones16 = jnp.ones((LCH, 16), jnp.float32)

    x_src = _sc_gather(x, src_p, nch=NCH_E, width=F)
    msg1 = _edge1(ea_p, x_src, We1a, be1a2, W1r, b1r, S1)
    agg1 = _sc_scatter_add(msg1, dst_p, z1, nch=NCH_E, width=M1_OUT, npad=NP1)
    out1 = _node1(x, Wr1, agg1, b1_2)

    x_src2 = _sc_gather(out1, src_p, nch=NCH_E, width=M1_OUT)
    msg2 = _edge2(ea_p, x_src2, We2a, be2a2, W2r, b2r, S2)
    agg2 = _sc_scatter_add(msg2, dst_p, z2, nch=NCH_E, width=M2_OUT, npad=NP1)
    h, x_1 = _node2(out1, Wr2, agg2, b2_2, batch2d)

    hsum, hcnt = _sc_pool2(h, a0_p, a1_p, ones16, z64, z16)

    t4, s4 = _convpre(hsum[0:N2], hsum[NP2:NP2 + N2], hcnt[0:N2],
                      hcnt[NP2:NP2 + N2], iso_type_2, W4n, W4s, b4_2)
    agg4 = _sc_gather_scatter(t4, src2_p, dst2_p, z64, nch=NCH_E, width=D, npad=NP2)
    t5, s5 = _convmid(s4, agg4, W5n, W5s, b5_2)
    agg5 = _sc_gather_scatter(t5, src2_p, dst2_p, z64, nch=NCH_E, width=D, npad=NP2)

    batch2_3d = batch_2.astype(i32).reshape(N2 // PB, 1, PB)
    x2s, x2c = _pool2b(s5, agg5[0:N2], agg5[NP2:NP2 + N2], batch2_3d)
    o = _readout(x_1, x2s, x2c, Wo1, bo1_2, Wo2, bo2_2, Wo3, bo3_2)
    return o.reshape(-1)
